# Initial kernel scaffold; baseline (speedup 1.0000x reference)
import jax
import jax.numpy as jnp
from jax.experimental import pallas as pl


def kernel(hidden_states, gate_w, w_gate, w_up, w_down):
    raise NotImplementedError("write your pallas kernel here")


# speedup vs baseline: 1.7543x; 1.7543x over previous
import functools

import jax
import jax.numpy as jnp
from jax import lax
from jax.experimental import pallas as pl
from jax.experimental.pallas import tpu as pltpu
from jax.experimental.pallas import tpu_sc as plsc

E = 8
TOPK = 2
D = 1024
F = 2816
T = 2048
BLK = 128
NB = T * TOPK // BLK + E
P = NB * BLK
FC = 2
F2 = F // FC

NC, NS = 2, 16
NW = NC * NS
TPW = T // NW


def _router_body(x_ref, gw_ref, logits_ref, pos_ref, wrep_ref, meta_ref):
    x = x_ref[...]
    gw = gw_ref[...]
    logits = jnp.dot(x, gw, preferred_element_type=jnp.float32)
    logits_ref[...] = logits

    lane = lax.broadcasted_iota(jnp.int32, (T, E), 1)
    neg = jnp.float32(-1e30)
    m1 = jnp.max(logits, axis=1, keepdims=True)
    i1 = jnp.min(jnp.where(logits == m1, lane, E), axis=1, keepdims=True)
    sel1 = lane == i1
    l2 = jnp.where(sel1, neg, logits)
    m2 = jnp.max(l2, axis=1, keepdims=True)
    i2 = jnp.min(jnp.where(l2 == m2, lane, E), axis=1, keepdims=True)
    sel2 = lane == i2

    wA = 1.0 / (1.0 + jnp.exp(m2 - m1))
    wB = 1.0 - wA

    m = sel1.astype(jnp.float32) + sel2.astype(jnp.float32)
    inc = m
    sh = 1
    while sh < T:
        inc = inc + jnp.concatenate(
            [jnp.zeros((sh, E), jnp.float32), inc[: T - sh, :]], axis=0)
        sh *= 2
    s_excl = inc - m
    counts = inc[T - 1: T, :]
    pc = jnp.ceil(counts / BLK) * BLK

    ii = lax.broadcasted_iota(jnp.int32, (E, E), 0)
    jj = lax.broadcasted_iota(jnp.int32, (E, E), 1)
    triu = (ii < jj).astype(jnp.float32)
    goff_row = jnp.dot(pc, triu, preferred_element_type=jnp.float32)

    dest = goff_row + s_excl
    pos0 = jnp.sum(jnp.where(sel1, dest, 0.0), axis=1, keepdims=True)
    pos1 = jnp.sum(jnp.where(sel2, dest, 0.0), axis=1, keepdims=True)
    pos_ref[...] = jnp.where(
        lane == 0, pos0, jnp.where(lane == 1, pos1, 0.0)).astype(jnp.int32)

    lane32 = lax.broadcasted_iota(jnp.int32, (T, 32), 1)
    wrep_ref[...] = jnp.where(lane32 < 16, wA, wB)

    eye = (ii == jj).astype(jnp.float32)
    pc_col = jnp.sum(jnp.dot(jnp.ones((E, 1), jnp.float32), pc,
                             preferred_element_type=jnp.float32) * eye,
                     axis=1, keepdims=True)
    tril = (jj < ii).astype(jnp.float32)
    goff_col = jnp.dot(tril, pc_col, preferred_element_type=jnp.float32)
    bstart = (lax.broadcasted_iota(jnp.int32, (E, 64), 1) * BLK
              ).astype(jnp.float32)
    cnt = jnp.sum((goff_col <= bstart).astype(jnp.float32), axis=0,
                  keepdims=True)
    be = jnp.maximum(cnt - 1.0, 0.0)
    meta_ref[...] = jnp.broadcast_to(be, (E, 64)).astype(jnp.int32)


def _router(x, gate_w):
    return pl.pallas_call(
        _router_body,
        out_shape=(
            jax.ShapeDtypeStruct((T, E), jnp.float32),
            jax.ShapeDtypeStruct((T, E), jnp.int32),
            jax.ShapeDtypeStruct((T, 32), jnp.float32),
            jax.ShapeDtypeStruct((E, 64), jnp.int32),
        ),
    )(x, gate_w)


def _dispatch_body(x_hbm, p0_hbm, p1_hbm, xs_hbm, xbuf, p0v, p1v, sem0, sem1):
    w = lax.axis_index("s") * NC + lax.axis_index("c")
    pltpu.sync_copy(x_hbm.at[pl.ds(w * TPW, TPW)], xbuf)
    pltpu.sync_copy(p0_hbm.at[pl.ds(w, 1)], p0v)
    pltpu.sync_copy(p1_hbm.at[pl.ds(w, 1)], p1v)
    c0 = pltpu.async_copy(xbuf, xs_hbm.at[p0v.at[0]], sem0)
    c1 = pltpu.async_copy(xbuf, xs_hbm.at[p1v.at[0]], sem1)
    c0.wait()
    c1.wait()


def _dispatch(x, pos0, pos1):
    mesh = plsc.VectorSubcoreMesh(core_axis_name="c", subcore_axis_name="s",
                                  num_cores=NC, num_subcores=NS)
    return pl.kernel(
        _dispatch_body,
        out_type=jax.ShapeDtypeStruct((P, D), jnp.float32),
        mesh=mesh,
        scratch_types=[
            pltpu.VMEM((TPW, D), jnp.float32),
            pltpu.VMEM((1, TPW), jnp.int32),
            pltpu.VMEM((1, TPW), jnp.int32),
            pltpu.SemaphoreType.DMA,
            pltpu.SemaphoreType.DMA,
        ],
    )(x, pos0, pos1)


def _mlp_up_body(s_ref, xs_ref, wg_ref, wu_ref, h_ref):
    xb = xs_ref[...].astype(jnp.bfloat16)
    g = jnp.dot(xb, wg_ref[0].astype(jnp.bfloat16),
                preferred_element_type=jnp.float32)
    u = jnp.dot(xb, wu_ref[0].astype(jnp.bfloat16),
                preferred_element_type=jnp.float32)
    h_ref[...] = (g * (1.0 / (1.0 + jnp.exp(-g))) * u).astype(jnp.bfloat16)


def _mlp_up(be, xs, w_gate, w_up):
    grid_spec = pltpu.PrefetchScalarGridSpec(
        num_scalar_prefetch=1,
        grid=(FC, NB),
        in_specs=[
            pl.BlockSpec((BLK, D), lambda f, b, s: (b, 0)),
            pl.BlockSpec((1, D, F2), lambda f, b, s: (s[b], 0, f)),
            pl.BlockSpec((1, D, F2), lambda f, b, s: (s[b], 0, f)),
        ],
        out_specs=pl.BlockSpec((BLK, F2), lambda f, b, s: (b, f)),
    )
    return pl.pallas_call(
        _mlp_up_body,
        grid_spec=grid_spec,
        out_shape=jax.ShapeDtypeStruct((P, F), jnp.bfloat16),
        compiler_params=pltpu.CompilerParams(
            dimension_semantics=("arbitrary", "arbitrary")),
    )(be, xs, w_gate, w_up)


def _mlp_down_body(s_ref, h_ref, wd_ref, out_ref):
    out_ref[...] = jnp.dot(h_ref[...], wd_ref[0].astype(jnp.bfloat16),
                           preferred_element_type=jnp.float32)


def _mlp_down(be, h, w_down):
    grid_spec = pltpu.PrefetchScalarGridSpec(
        num_scalar_prefetch=1,
        grid=(NB,),
        in_specs=[
            pl.BlockSpec((BLK, F), lambda b, s: (b, 0)),
            pl.BlockSpec((1, F, D), lambda b, s: (s[b], 0, 0)),
        ],
        out_specs=pl.BlockSpec((BLK, D), lambda b, s: (b, 0)),
    )
    return pl.pallas_call(
        _mlp_down_body,
        grid_spec=grid_spec,
        out_shape=jax.ShapeDtypeStruct((P, D), jnp.float32),
        compiler_params=pltpu.CompilerParams(
            dimension_semantics=("arbitrary",)),
    )(be, h, w_down)


def _combine_body(outs_hbm, p0_hbm, p1_hbm, w0_hbm, w1_hbm, fin_hbm,
                  p0v, p1v, w0buf, w1buf, buf0, buf1, res, sem0, sem1):
    w = lax.axis_index("s") * NC + lax.axis_index("c")
    pltpu.sync_copy(p0_hbm.at[pl.ds(w, 1)], p0v)
    pltpu.sync_copy(p1_hbm.at[pl.ds(w, 1)], p1v)
    pltpu.sync_copy(w0_hbm.at[pl.ds(w * TPW * 16, TPW * 16)], w0buf)
    pltpu.sync_copy(w1_hbm.at[pl.ds(w * TPW * 16, TPW * 16)], w1buf)

    def half_step(half, _):
        c0 = pltpu.async_copy(outs_hbm.at[p0v.at[0, half]], buf0, sem0)
        c1 = pltpu.async_copy(outs_hbm.at[p1v.at[0, half]], buf1, sem1)
        c0.wait()
        c1.wait()

        def row_step(i, _):
            w0v = w0buf[pl.ds(half * 512 + i * 16, 16)]
            w1v = w1buf[pl.ds(half * 512 + i * 16, 16)]

            def chunk_step(j, _):
                res[i, pl.ds(j * 16, 16)] = (
                    w0v * buf0[i, pl.ds(j * 16, 16)]
                    + w1v * buf1[i, pl.ds(j * 16, 16)])
                return 0

            lax.fori_loop(0, D // 16, chunk_step, 0)
            return 0

        lax.fori_loop(0, 32, row_step, 0)
        pltpu.sync_copy(res, fin_hbm.at[pl.ds(w * TPW + half * 32, 32)])
        return 0

    lax.fori_loop(0, 2, half_step, 0)


def _combine(outs, pos0, pos1, w0f, w1f):
    mesh = plsc.VectorSubcoreMesh(core_axis_name="c", subcore_axis_name="s",
                                  num_cores=NC, num_subcores=NS)
    return pl.kernel(
        _combine_body,
        out_type=jax.ShapeDtypeStruct((T, D), jnp.float32),
        mesh=mesh,
        scratch_types=[
            pltpu.VMEM((1, 2, 32), jnp.int32),
            pltpu.VMEM((1, 2, 32), jnp.int32),
            pltpu.VMEM((TPW * 16,), jnp.float32),
            pltpu.VMEM((TPW * 16,), jnp.float32),
            pltpu.VMEM((32, D), jnp.float32),
            pltpu.VMEM((32, D), jnp.float32),
            pltpu.VMEM((32, D), jnp.float32),
            pltpu.SemaphoreType.DMA,
            pltpu.SemaphoreType.DMA,
        ],
    )(outs, pos0, pos1, w0f, w1f)


def kernel(hidden_states, gate_w, w_gate, w_up, w_down):
    B, S, _ = hidden_states.shape
    x = hidden_states.reshape(T, D)
    logits, pos, wrep, meta = _router(x, gate_w)
    be = meta[0, :NB]
    pos0 = pos[:, 0].reshape(NW, TPW)
    pos1 = pos[:, 1].reshape(NW, TPW)
    p0h = pos0.reshape(NW, 2, TPW // 2)
    p1h = pos1.reshape(NW, 2, TPW // 2)
    w0f = wrep[:, :16].reshape(-1)
    w1f = wrep[:, 16:].reshape(-1)

    xs = _dispatch(x, pos0, pos1)
    h = _mlp_up(be, xs, w_gate, w_up)
    outs = _mlp_down(be, h, w_down)
    final = _combine(outs, p0h, p1h, w0f, w1f)
    return final.reshape(B, S, D), logits
